# baseline (speedup 1.0000x reference)
import jax
import jax.numpy as jnp
from jax.experimental import pallas as pl

_NEG_RATIO = 3
_ALPHA = 1.5
_BETA = 1.0
_GAMMA = 2.0
_HI_BITS = 0x41800000


def _reg_kernel(rm_ref, tg_ref, pos_ref, e_ref, iw_ref, num_ref, psum_ref):
    e = e_ref[...]
    iw = iw_ref[...]
    th = rm_ref.shape[2]
    w = rm_ref.shape[3]
    acc = jnp.zeros((w, 7 * e.shape[0]), jnp.float32)
    for i in range(th):
        rm_row = rm_ref[0, :, i, :]
        rm_t = jax.lax.dot_general(
            iw, rm_row, (((1,), (1,)), ((), ())),
            preferred_element_type=jnp.float32,
            precision=jax.lax.Precision.HIGHEST)
        tg_row = tg_ref[0, i]
        mask = jnp.dot(pos_ref[0, i], e,
                       preferred_element_type=jnp.float32,
                       precision=jax.lax.Precision.HIGHEST)
        d = (rm_t - tg_row) * mask
        ad = jnp.abs(d)
        acc = acc + jnp.where(ad < 1.0, 0.5 * d * d, ad - 0.5)
    num_ref[...] = jnp.broadcast_to(jnp.sum(acc), (1, 1, 1, 1))
    psum_ref[...] = jnp.broadcast_to(jnp.sum(pos_ref[...]), (1, 1, 1, 1))


def _cls_kernel(psm_ref, pos_ref, neg_ref, clsp_ref, topk_ref, k_ref):
    x = psm_ref[...]
    pos = pos_ref[...]
    neg = neg_ref[...]
    p = jax.nn.sigmoid(x)
    clsp = jnp.sum(-pos * jnp.log(p + 1e-6))
    fpos = jnp.sum(pos)
    n = jnp.int32(x.size)
    k = jnp.minimum((_NEG_RATIO * (fpos + 1.0)).astype(jnp.int32), n)
    fl = jnp.maximum(-neg * jnp.log(1.0 - p + 1e-6), 0.0)
    bits = jax.lax.bitcast_convert_type(fl, jnp.int32)

    def body(_, carry):
        lo, hi = carry
        mid = lo + (hi - lo) // 2
        c = jnp.sum((bits > mid).astype(jnp.int32))
        shrink = c < k
        return (jnp.where(shrink, lo, mid + 1),
                jnp.where(shrink, mid, hi))

    lo, _ = jax.lax.fori_loop(
        0, 31, body, (jnp.int32(0), jnp.int32(_HI_BITS)))
    t = jax.lax.bitcast_convert_type(lo, jnp.float32)
    gt = bits > lo
    cnt = jnp.sum(gt.astype(jnp.int32))
    topk = jnp.sum(jnp.where(gt, fl, 0.0)) + (k - cnt).astype(jnp.float32) * t
    clsp_ref[...] = jnp.broadcast_to(clsp, (1, 1, 1))
    topk_ref[...] = jnp.broadcast_to(topk, (1, 1, 1))
    k_ref[...] = jnp.broadcast_to(k.astype(jnp.float32), (1, 1, 1))


def kernel(rm, psm, pos_equal_one, neg_equal_one, targets):
    b, a, h, w = psm.shape
    nframe = a // 2
    pos_cf = jnp.transpose(pos_equal_one, (0, 3, 1, 2))
    neg_cf = jnp.transpose(neg_equal_one, (0, 3, 1, 2))
    e = (jnp.arange(7 * a, dtype=jnp.int32)[None, :] // 7
         == jnp.arange(a, dtype=jnp.int32)[:, None]).astype(jnp.float32)
    iw = jnp.eye(w, dtype=jnp.float32)

    nh = 5
    th = h // nh
    num, psum = pl.pallas_call(
        _reg_kernel,
        grid=(b, nh),
        in_specs=[
            pl.BlockSpec((1, 7 * a, th, w), lambda i, j: (i, 0, j, 0)),
            pl.BlockSpec((1, th, w, 7 * a), lambda i, j: (i, j, 0, 0)),
            pl.BlockSpec((1, th, w, a), lambda i, j: (i, j, 0, 0)),
            pl.BlockSpec((a, 7 * a), lambda i, j: (0, 0)),
            pl.BlockSpec((w, w), lambda i, j: (0, 0)),
        ],
        out_specs=[
            pl.BlockSpec((1, 1, 1, 1), lambda i, j: (i, j, 0, 0)),
            pl.BlockSpec((1, 1, 1, 1), lambda i, j: (i, j, 0, 0)),
        ],
        out_shape=[
            jax.ShapeDtypeStruct((b, nh, 1, 1), jnp.float32),
            jax.ShapeDtypeStruct((b, nh, 1, 1), jnp.float32),
        ],
    )(rm, targets, pos_equal_one, e, iw)

    nf = b * nframe
    clsp, topk, kf = pl.pallas_call(
        _cls_kernel,
        grid=(nf,),
        in_specs=[
            pl.BlockSpec((1, 2, h, w), lambda f: (f // nframe, f % nframe, 0, 0)),
            pl.BlockSpec((1, 2, h, w), lambda f: (f // nframe, f % nframe, 0, 0)),
            pl.BlockSpec((1, 2, h, w), lambda f: (f // nframe, f % nframe, 0, 0)),
        ],
        out_specs=[
            pl.BlockSpec((1, 1, 1), lambda f: (f, 0, 0)),
            pl.BlockSpec((1, 1, 1), lambda f: (f, 0, 0)),
            pl.BlockSpec((1, 1, 1), lambda f: (f, 0, 0)),
        ],
        out_shape=[
            jax.ShapeDtypeStruct((nf, 1, 1), jnp.float32),
            jax.ShapeDtypeStruct((nf, 1, 1), jnp.float32),
            jax.ShapeDtypeStruct((nf, 1, 1), jnp.float32),
        ],
    )(psm, pos_cf, neg_cf)

    pos_sum = jnp.sum(psum)
    reg_loss = _GAMMA * jnp.sum(num) / (pos_sum + 1e-6)
    cls_pos_loss = _ALPHA * jnp.sum(clsp) / (pos_sum + 1e-6)
    cls_neg_loss = _BETA * jnp.sum(topk) / (jnp.sum(kf) + 1e-6)
    conf_loss = cls_pos_loss + cls_neg_loss
    return (conf_loss, reg_loss, cls_pos_loss, cls_neg_loss)


# speedup vs baseline: 1.1924x; 1.1924x over previous
import jax
import jax.numpy as jnp
from jax.experimental import pallas as pl

_NEG_RATIO = 3
_ALPHA = 1.5
_BETA = 1.0
_GAMMA = 2.0
_HI_BITS = 0x41800000


def _reg_kernel(rm_ref, tg_ref, pos_ref, e_ref, num_ref, psum_ref):
    pos = pos_ref[...]
    th, w = pos.shape[1], pos.shape[2]
    rm_t = jnp.transpose(rm_ref[0], (1, 2, 0))
    mask = jnp.dot(pos.reshape(th * w, 10), e_ref[...],
                   preferred_element_type=jnp.float32).reshape(th, w, 70)
    d = (rm_t - tg_ref[0]) * mask
    ad = jnp.abs(d)
    sl1 = jnp.where(ad < 1.0, 0.5 * d * d, ad - 0.5)
    num_ref[...] = jnp.broadcast_to(jnp.sum(sl1), (1, 1, 1, 1))
    psum_ref[...] = jnp.broadcast_to(jnp.sum(pos), (1, 1, 1, 1))


def _cls_kernel(psm_ref, pos_ref, neg_ref, clsp_ref, topk_ref, k_ref):
    x = psm_ref[...]
    pos = pos_ref[...]
    neg = neg_ref[...]
    p = jax.nn.sigmoid(x)
    clsp = jnp.sum(-pos * jnp.log(p + 1e-6))
    fpos = jnp.sum(pos)
    n = jnp.int32(x.size)
    k = jnp.minimum((_NEG_RATIO * (fpos + 1.0)).astype(jnp.int32), n)
    fl = jnp.maximum(-neg * jnp.log(1.0 - p + 1e-6), 0.0)
    bits = jax.lax.bitcast_convert_type(fl, jnp.int32)

    def body(_, carry):
        lo, hi = carry
        mid = lo + (hi - lo) // 2
        c = jnp.sum((bits > mid).astype(jnp.int32))
        shrink = c < k
        return (jnp.where(shrink, lo, mid + 1),
                jnp.where(shrink, mid, hi))

    lo, _ = jax.lax.fori_loop(
        0, 31, body, (jnp.int32(0), jnp.int32(_HI_BITS)))
    t = jax.lax.bitcast_convert_type(lo, jnp.float32)
    gt = bits > lo
    cnt = jnp.sum(gt.astype(jnp.int32))
    topk = jnp.sum(jnp.where(gt, fl, 0.0)) + (k - cnt).astype(jnp.float32) * t
    clsp_ref[...] = jnp.broadcast_to(clsp, (1, 1, 1))
    topk_ref[...] = jnp.broadcast_to(topk, (1, 1, 1))
    k_ref[...] = jnp.broadcast_to(k.astype(jnp.float32), (1, 1, 1))


def kernel(rm, psm, pos_equal_one, neg_equal_one, targets):
    b, a, h, w = psm.shape
    nframe = a // 2
    pos_cf = jnp.transpose(pos_equal_one, (0, 3, 1, 2))
    neg_cf = jnp.transpose(neg_equal_one, (0, 3, 1, 2))
    e = (jnp.arange(7 * a, dtype=jnp.int32)[None, :] // 7
         == jnp.arange(a, dtype=jnp.int32)[:, None]).astype(jnp.float32)
    nh = 5
    th = h // nh
    num, psum = pl.pallas_call(
        _reg_kernel,
        grid=(b, nh),
        in_specs=[
            pl.BlockSpec((1, 7 * a, th, w), lambda i, j: (i, 0, j, 0)),
            pl.BlockSpec((1, th, w, 7 * a), lambda i, j: (i, j, 0, 0)),
            pl.BlockSpec((1, th, w, a), lambda i, j: (i, j, 0, 0)),
            pl.BlockSpec((a, 7 * a), lambda i, j: (0, 0)),
        ],
        out_specs=[
            pl.BlockSpec((1, 1, 1, 1), lambda i, j: (i, j, 0, 0)),
            pl.BlockSpec((1, 1, 1, 1), lambda i, j: (i, j, 0, 0)),
        ],
        out_shape=[
            jax.ShapeDtypeStruct((b, nh, 1, 1), jnp.float32),
            jax.ShapeDtypeStruct((b, nh, 1, 1), jnp.float32),
        ],
    )(rm, targets, pos_equal_one, e)

    nf = b * nframe
    clsp, topk, kf = pl.pallas_call(
        _cls_kernel,
        grid=(nf,),
        in_specs=[
            pl.BlockSpec((1, 2, h, w), lambda f: (f // nframe, f % nframe, 0, 0)),
            pl.BlockSpec((1, 2, h, w), lambda f: (f // nframe, f % nframe, 0, 0)),
            pl.BlockSpec((1, 2, h, w), lambda f: (f // nframe, f % nframe, 0, 0)),
        ],
        out_specs=[
            pl.BlockSpec((1, 1, 1), lambda f: (f, 0, 0)),
            pl.BlockSpec((1, 1, 1), lambda f: (f, 0, 0)),
            pl.BlockSpec((1, 1, 1), lambda f: (f, 0, 0)),
        ],
        out_shape=[
            jax.ShapeDtypeStruct((nf, 1, 1), jnp.float32),
            jax.ShapeDtypeStruct((nf, 1, 1), jnp.float32),
            jax.ShapeDtypeStruct((nf, 1, 1), jnp.float32),
        ],
    )(psm, pos_cf, neg_cf)

    pos_sum = jnp.sum(psum)
    reg_loss = _GAMMA * jnp.sum(num) / (pos_sum + 1e-6)
    cls_pos_loss = _ALPHA * jnp.sum(clsp) / (pos_sum + 1e-6)
    cls_neg_loss = _BETA * jnp.sum(topk) / (jnp.sum(kf) + 1e-6)
    conf_loss = cls_pos_loss + cls_neg_loss
    return (conf_loss, reg_loss, cls_pos_loss, cls_neg_loss)


# speedup vs baseline: 1.2894x; 1.0814x over previous
import jax
import jax.numpy as jnp
from jax.experimental import pallas as pl

_NEG_RATIO = 3
_ALPHA = 1.5
_BETA = 1.0
_GAMMA = 2.0
_HI_BITS = 0x41800000


def _reg_kernel(rm_ref, tg_ref, pos_ref, e_ref, num_ref, psum_ref):
    rm = rm_ref[...]
    tg = tg_ref[...]
    pos = pos_ref[...]
    th, w = pos.shape[1], pos.shape[2]
    mask = jnp.dot(pos.reshape(th * w, 10), e_ref[...],
                   preferred_element_type=jnp.float32)
    d = (rm - tg).reshape(th * w, 70) * mask
    ad = jnp.abs(d)
    sl1 = jnp.where(ad < 1.0, 0.5 * d * d, ad - 0.5)
    num_ref[...] = jnp.broadcast_to(jnp.sum(sl1), (1, 1, 1, 1))
    psum_ref[...] = jnp.broadcast_to(jnp.sum(pos), (1, 1, 1, 1))


def _cls_kernel(psm_ref, pos_ref, neg_ref, clsp_ref, topk_ref, k_ref):
    x = psm_ref[...]
    pos = pos_ref[...]
    neg = neg_ref[...]
    nf = x.shape[0]
    p = jax.nn.sigmoid(x)
    clsp = jnp.sum(-pos * jnp.log(p + 1e-6))
    fpos = jnp.sum(pos, axis=(1, 2))
    n = jnp.int32(x.shape[1] * x.shape[2])
    k = jnp.minimum((_NEG_RATIO * (fpos + 1.0)).astype(jnp.int32), n)
    fl = jnp.maximum(-neg * jnp.log(1.0 - p + 1e-6), 0.0)
    bits = jax.lax.bitcast_convert_type(fl, jnp.int32)

    def body(_, carry):
        lo, hi = carry
        mid = lo + (hi - lo) // 2
        c = jnp.sum((bits > mid[:, None, None]).astype(jnp.int32),
                    axis=(1, 2))
        shrink = c < k
        return (jnp.where(shrink, lo, mid + 1),
                jnp.where(shrink, mid, hi))

    lo, _ = jax.lax.fori_loop(
        0, 31, body,
        (jnp.zeros((nf,), jnp.int32), jnp.full((nf,), _HI_BITS, jnp.int32)))
    t = jax.lax.bitcast_convert_type(lo, jnp.float32)
    gt = bits > lo[:, None, None]
    cnt = jnp.sum(gt.astype(jnp.int32), axis=(1, 2))
    topk = (jnp.sum(jnp.where(gt, fl, 0.0), axis=(1, 2))
            + (k - cnt).astype(jnp.float32) * t)
    pad = clsp_ref.shape[1] - nf
    clsp_ref[...] = jnp.broadcast_to(clsp, clsp_ref.shape)
    topk_ref[...] = jnp.pad(topk, (0, pad)).reshape(1, nf + pad)
    k_ref[...] = jnp.pad(k.astype(jnp.float32), (0, pad)).reshape(1, nf + pad)


def kernel(rm, psm, pos_equal_one, neg_equal_one, targets):
    b, a, h, w = psm.shape
    nframe = a // 2
    nf = b * nframe
    fr = 2 * h * w // 128
    rm_t = jnp.transpose(rm, (0, 2, 3, 1))
    psm_r = jnp.reshape(psm, (nf, fr, 128))
    pos_r = jnp.reshape(jnp.transpose(pos_equal_one, (0, 3, 1, 2)),
                        (nf, fr, 128))
    neg_r = jnp.reshape(jnp.transpose(neg_equal_one, (0, 3, 1, 2)),
                        (nf, fr, 128))
    e = (jnp.arange(7 * a, dtype=jnp.int32)[None, :] // 7
         == jnp.arange(a, dtype=jnp.int32)[:, None]).astype(jnp.float32)
    nh = 8
    th = h // nh
    num, psum = pl.pallas_call(
        _reg_kernel,
        grid=(b, nh),
        in_specs=[
            pl.BlockSpec((1, th, w, 7 * a), lambda i, j: (i, j, 0, 0)),
            pl.BlockSpec((1, th, w, 7 * a), lambda i, j: (i, j, 0, 0)),
            pl.BlockSpec((1, th, w, a), lambda i, j: (i, j, 0, 0)),
            pl.BlockSpec((a, 7 * a), lambda i, j: (0, 0)),
        ],
        out_specs=[
            pl.BlockSpec((1, 1, 1, 1), lambda i, j: (i, j, 0, 0)),
            pl.BlockSpec((1, 1, 1, 1), lambda i, j: (i, j, 0, 0)),
        ],
        out_shape=[
            jax.ShapeDtypeStruct((b, nh, 1, 1), jnp.float32),
            jax.ShapeDtypeStruct((b, nh, 1, 1), jnp.float32),
        ],
    )(rm_t, targets, pos_equal_one, e)

    clsp, topk, kf = pl.pallas_call(
        _cls_kernel,
        out_shape=[
            jax.ShapeDtypeStruct((1, 32), jnp.float32),
            jax.ShapeDtypeStruct((1, 32), jnp.float32),
            jax.ShapeDtypeStruct((1, 32), jnp.float32),
        ],
    )(psm_r, pos_r, neg_r)

    pos_sum = jnp.sum(psum)
    reg_loss = _GAMMA * jnp.sum(num) / (pos_sum + 1e-6)
    cls_pos_loss = _ALPHA * clsp[0, 0] / (pos_sum + 1e-6)
    cls_neg_loss = _BETA * jnp.sum(topk[0, :nf]) / (jnp.sum(kf[0, :nf]) + 1e-6)
    conf_loss = cls_pos_loss + cls_neg_loss
    return (conf_loss, reg_loss, cls_pos_loss, cls_neg_loss)


# speedup vs baseline: 1.2914x; 1.0015x over previous
import jax
import jax.numpy as jnp
from jax.experimental import pallas as pl

_NEG_RATIO = 3
_ALPHA = 1.5
_BETA = 1.0
_GAMMA = 2.0
_HI_BITS = 0x41800000


def _reg_kernel(rm_ref, tg_ref, pos_ref, e_ref, num_ref, psum_ref):
    rm = rm_ref[...]
    tg = tg_ref[...]
    pos = pos_ref[...]
    th, w = pos.shape[1], pos.shape[2]
    mask = jnp.dot(pos.reshape(th * w, 10), e_ref[...],
                   preferred_element_type=jnp.float32)
    d = (rm - tg).reshape(th * w, 70) * mask
    ad = jnp.abs(d)
    sl1 = jnp.where(ad < 1.0, 0.5 * d * d, ad - 0.5)
    num_ref[...] = jnp.broadcast_to(jnp.sum(sl1), (1, 1, 1, 1))
    psum_ref[...] = jnp.broadcast_to(jnp.sum(pos), (1, 1, 1, 1))


def _cls_kernel(psm_ref, pos_ref, neg_ref, clsp_ref, topk_ref, k_ref):
    x = psm_ref[...]
    pos = pos_ref[...]
    neg = neg_ref[...]
    p = jax.nn.sigmoid(x)
    clsp = jnp.sum(-pos * jnp.log(p + 1e-6))
    fpos = jnp.sum(pos)
    n = jnp.int32(x.size)
    k = jnp.minimum((_NEG_RATIO * (fpos + 1.0)).astype(jnp.int32), n)
    fl = jnp.maximum(-neg * jnp.log(1.0 - p + 1e-6), 0.0)
    bits = jax.lax.bitcast_convert_type(fl, jnp.int32)

    def body(_, carry):
        lo, hi = carry
        mid = lo + (hi - lo) // 2
        c = jnp.sum((bits > mid).astype(jnp.int32))
        shrink = c < k
        return (jnp.where(shrink, lo, mid + 1),
                jnp.where(shrink, mid, hi))

    lo, _ = jax.lax.fori_loop(
        0, 31, body, (jnp.int32(0), jnp.int32(_HI_BITS)))
    t = jax.lax.bitcast_convert_type(lo, jnp.float32)
    gt = bits > lo
    cnt = jnp.sum(gt.astype(jnp.int32))
    topk = jnp.sum(jnp.where(gt, fl, 0.0)) + (k - cnt).astype(jnp.float32) * t
    clsp_ref[...] = jnp.broadcast_to(clsp, (1, 1, 1))
    topk_ref[...] = jnp.broadcast_to(topk, (1, 1, 1))
    k_ref[...] = jnp.broadcast_to(k.astype(jnp.float32), (1, 1, 1))


def kernel(rm, psm, pos_equal_one, neg_equal_one, targets):
    b, a, h, w = psm.shape
    nframe = a // 2
    nf = b * nframe
    fr = 2 * h * w // 128
    rm_t = jnp.transpose(rm, (0, 2, 3, 1))
    psm_r = jnp.reshape(psm, (nf, fr, 128))
    pos_r = jnp.reshape(jnp.transpose(pos_equal_one, (0, 3, 1, 2)),
                        (nf, fr, 128))
    neg_r = jnp.reshape(jnp.transpose(neg_equal_one, (0, 3, 1, 2)),
                        (nf, fr, 128))
    e = (jnp.arange(7 * a, dtype=jnp.int32)[None, :] // 7
         == jnp.arange(a, dtype=jnp.int32)[:, None]).astype(jnp.float32)
    nh = 8
    th = h // nh
    num, psum = pl.pallas_call(
        _reg_kernel,
        grid=(b, nh),
        in_specs=[
            pl.BlockSpec((1, th, w, 7 * a), lambda i, j: (i, j, 0, 0)),
            pl.BlockSpec((1, th, w, 7 * a), lambda i, j: (i, j, 0, 0)),
            pl.BlockSpec((1, th, w, a), lambda i, j: (i, j, 0, 0)),
            pl.BlockSpec((a, 7 * a), lambda i, j: (0, 0)),
        ],
        out_specs=[
            pl.BlockSpec((1, 1, 1, 1), lambda i, j: (i, j, 0, 0)),
            pl.BlockSpec((1, 1, 1, 1), lambda i, j: (i, j, 0, 0)),
        ],
        out_shape=[
            jax.ShapeDtypeStruct((b, nh, 1, 1), jnp.float32),
            jax.ShapeDtypeStruct((b, nh, 1, 1), jnp.float32),
        ],
    )(rm_t, targets, pos_equal_one, e)

    clsp, topk, kf = pl.pallas_call(
        _cls_kernel,
        grid=(nf,),
        in_specs=[
            pl.BlockSpec((1, fr, 128), lambda f: (f, 0, 0)),
            pl.BlockSpec((1, fr, 128), lambda f: (f, 0, 0)),
            pl.BlockSpec((1, fr, 128), lambda f: (f, 0, 0)),
        ],
        out_specs=[
            pl.BlockSpec((1, 1, 1), lambda f: (f, 0, 0)),
            pl.BlockSpec((1, 1, 1), lambda f: (f, 0, 0)),
            pl.BlockSpec((1, 1, 1), lambda f: (f, 0, 0)),
        ],
        out_shape=[
            jax.ShapeDtypeStruct((nf, 1, 1), jnp.float32),
            jax.ShapeDtypeStruct((nf, 1, 1), jnp.float32),
            jax.ShapeDtypeStruct((nf, 1, 1), jnp.float32),
        ],
    )(psm_r, pos_r, neg_r)

    pos_sum = jnp.sum(psum)
    reg_loss = _GAMMA * jnp.sum(num) / (pos_sum + 1e-6)
    cls_pos_loss = _ALPHA * jnp.sum(clsp) / (pos_sum + 1e-6)
    cls_neg_loss = _BETA * jnp.sum(topk) / (jnp.sum(kf) + 1e-6)
    conf_loss = cls_pos_loss + cls_neg_loss
    return (conf_loss, reg_loss, cls_pos_loss, cls_neg_loss)


# speedup vs baseline: 1.4337x; 1.1102x over previous
import jax
import jax.numpy as jnp
from jax.experimental import pallas as pl

_NEG_RATIO = 3
_ALPHA = 1.5
_BETA = 1.0
_GAMMA = 2.0
_HI_BITS = 0x41800000


def _reg_kernel(rm_ref, tg_ref, pos_ref, e_ref, num_ref, psum_ref):
    rm = rm_ref[...]
    tg = tg_ref[...]
    pos = pos_ref[...]
    th, w = pos.shape[1], pos.shape[2]
    mask = jnp.dot(pos.reshape(th * w, 10), e_ref[...],
                   preferred_element_type=jnp.float32)
    d = (rm - tg).reshape(th * w, 70) * mask
    ad = jnp.abs(d)
    sl1 = jnp.where(ad < 1.0, 0.5 * d * d, ad - 0.5)
    num_ref[...] = jnp.broadcast_to(jnp.sum(sl1), (1, 1, 1, 1))
    psum_ref[...] = jnp.broadcast_to(jnp.sum(pos), (1, 1, 1, 1))


def _cls_kernel(psm_ref, pos_ref, neg_ref, clsp_ref, topk_ref, k_ref):
    x = psm_ref[...]
    pos = pos_ref[...]
    neg = neg_ref[...]
    p = jax.nn.sigmoid(x)
    clsp = jnp.sum(-pos * jnp.log(p + 1e-6))
    fpos = jnp.sum(pos)
    n = jnp.int32(x.size)
    k = jnp.minimum((_NEG_RATIO * (fpos + 1.0)).astype(jnp.int32), n)
    fl = jnp.maximum(-neg * jnp.log(1.0 - p + 1e-6), 0.0)
    bits = jax.lax.bitcast_convert_type(fl, jnp.int32)

    def body(_, carry):
        lo, hi = carry
        mid = lo + (hi - lo) // 2
        c = jnp.sum((bits > mid).astype(jnp.int32))
        shrink = c < k
        return (jnp.where(shrink, lo, mid + 1),
                jnp.where(shrink, mid, hi))

    lo, _ = jax.lax.fori_loop(
        0, 31, body, (jnp.int32(0), jnp.int32(_HI_BITS)))
    t = jax.lax.bitcast_convert_type(lo, jnp.float32)
    gt = bits > lo
    cnt = jnp.sum(gt.astype(jnp.int32))
    topk = jnp.sum(jnp.where(gt, fl, 0.0)) + (k - cnt).astype(jnp.float32) * t
    clsp_ref[...] = jnp.broadcast_to(clsp, (1, 1, 1))
    topk_ref[...] = jnp.broadcast_to(topk, (1, 1, 1))
    k_ref[...] = jnp.broadcast_to(k.astype(jnp.float32), (1, 1, 1))


def kernel(rm, psm, pos_equal_one, neg_equal_one, targets):
    b, a, h, w = psm.shape
    nframe = a // 2
    nf = b * nframe
    rm_t = jnp.transpose(rm, (0, 2, 3, 1))
    pos_cf = jnp.transpose(pos_equal_one, (0, 3, 1, 2))
    neg_cf = jnp.transpose(neg_equal_one, (0, 3, 1, 2))
    e = (jnp.arange(7 * a, dtype=jnp.int32)[None, :] // 7
         == jnp.arange(a, dtype=jnp.int32)[:, None]).astype(jnp.float32)
    nh = 8
    th = h // nh
    num, psum = pl.pallas_call(
        _reg_kernel,
        grid=(b, nh),
        in_specs=[
            pl.BlockSpec((1, th, w, 7 * a), lambda i, j: (i, j, 0, 0)),
            pl.BlockSpec((1, th, w, 7 * a), lambda i, j: (i, j, 0, 0)),
            pl.BlockSpec((1, th, w, a), lambda i, j: (i, j, 0, 0)),
            pl.BlockSpec((a, 7 * a), lambda i, j: (0, 0)),
        ],
        out_specs=[
            pl.BlockSpec((1, 1, 1, 1), lambda i, j: (i, j, 0, 0)),
            pl.BlockSpec((1, 1, 1, 1), lambda i, j: (i, j, 0, 0)),
        ],
        out_shape=[
            jax.ShapeDtypeStruct((b, nh, 1, 1), jnp.float32),
            jax.ShapeDtypeStruct((b, nh, 1, 1), jnp.float32),
        ],
    )(rm_t, targets, pos_equal_one, e)

    clsp, topk, kf = pl.pallas_call(
        _cls_kernel,
        grid=(nf,),
        in_specs=[
            pl.BlockSpec((1, 2, h, w), lambda f: (f // 5, f % 5, 0, 0)),
            pl.BlockSpec((1, 2, h, w), lambda f: (f // 5, f % 5, 0, 0)),
            pl.BlockSpec((1, 2, h, w), lambda f: (f // 5, f % 5, 0, 0)),
        ],
        out_specs=[
            pl.BlockSpec((1, 1, 1), lambda f: (f, 0, 0)),
            pl.BlockSpec((1, 1, 1), lambda f: (f, 0, 0)),
            pl.BlockSpec((1, 1, 1), lambda f: (f, 0, 0)),
        ],
        out_shape=[
            jax.ShapeDtypeStruct((nf, 1, 1), jnp.float32),
            jax.ShapeDtypeStruct((nf, 1, 1), jnp.float32),
            jax.ShapeDtypeStruct((nf, 1, 1), jnp.float32),
        ],
    )(psm, pos_cf, neg_cf)

    pos_sum = jnp.sum(psum)
    reg_loss = _GAMMA * jnp.sum(num) / (pos_sum + 1e-6)
    cls_pos_loss = _ALPHA * jnp.sum(clsp) / (pos_sum + 1e-6)
    cls_neg_loss = _BETA * jnp.sum(topk) / (jnp.sum(kf) + 1e-6)
    conf_loss = cls_pos_loss + cls_neg_loss
    return (conf_loss, reg_loss, cls_pos_loss, cls_neg_loss)
